# baseline (speedup 1.0000x reference)
import functools

import jax
import jax.numpy as jnp
from jax import lax
from jax.experimental import pallas as pl
from jax.experimental.pallas import tpu as pltpu
from jax.experimental.pallas import tpu_sc as plsc

LANES = 128
CHUNK = 256
B_PER_W = 1024
SUB = 16


@functools.lru_cache(maxsize=None)
def _make_lookup(vocab, batch, hist, dim):
    info = plsc.get_sparse_core_info()
    nc, ns = info.num_cores, info.num_subcores
    h_half = hist // 2
    n_bw = batch // B_PER_W
    runs = B_PER_W // CHUNK
    n_chunks = h_half * runs
    per_w = h_half * B_PER_W
    sr = dim * 4
    streams = CHUNK // LANES

    mesh = plsc.VectorSubcoreMesh(core_axis_name="c", subcore_axis_name="s")

    @functools.partial(
        pl.kernel,
        mesh=mesh,
        out_type=jax.ShapeDtypeStruct((hist, dim, batch), jnp.float32),
        scratch_types=[
            pltpu.VMEM((per_w,), jnp.int32),
            pltpu.VMEM((streams, LANES), jnp.int32),
            pltpu.VMEM((streams, LANES), jnp.int32),
            pltpu.VMEM((CHUNK, sr), jnp.float32),
            pltpu.VMEM((CHUNK, sr), jnp.float32),
            pltpu.VMEM((dim, CHUNK), jnp.float32),
            pltpu.VMEM((dim, CHUNK), jnp.float32),
            pltpu.SemaphoreType.DMA,
            pltpu.SemaphoreType.DMA,
            pltpu.SemaphoreType.DMA,
            pltpu.SemaphoreType.DMA,
        ],
        compiler_params=pltpu.CompilerParams(use_tc_tiling_on_sc=True,
                                             needs_layout_passes=False),
    )
    def lookup(table_hbm, ids_hbm, out_hbm, idx_v, sb0, sb1, ch0, ch1,
               tr0, tr1, semg0, semg1, semt0, semt1):
        wid = lax.axis_index("s") * nc + lax.axis_index("c")
        hh = wid // n_bw
        bb = wid % n_bw
        b0 = bb * B_PER_W

        pltpu.sync_copy(ids_hbm.at[pl.ds(wid * per_w, per_w)], idx_v)

        def compute_sidx(g, sb):
            p0 = g * CHUNK
            for j in range(streams):
                for g8 in range(LANES // SUB):
                    v = idx_v[pl.ds(p0 + j * LANES + g8 * SUB, SUB)]
                    sb[j, pl.ds(g8 * SUB, SUB)] = v >> 2

        def fire_gather(sb, ch, semg):
            for j in range(streams):
                pltpu.async_copy(table_hbm.at[sb.at[j]],
                                 ch.at[pl.ds(j * LANES, LANES)], semg)

        def wait_gather(ch, semg):
            pltpu.make_async_copy(table_hbm.at[pl.ds(0, CHUNK)], ch,
                                  semg).wait()

        def extract(g, ch, tr):
            p0 = g * CHUNK
            ivec = lax.iota(jnp.int32, SUB)

            @plsc.parallel_loop(0, CHUNK // SUB, unroll=8)
            def _(g8):
                v = idx_v[pl.ds(p0 + g8 * SUB, SUB)]
                jbase = (v & 3) * dim
                rows = ivec + g8 * SUB
                for d in range(dim):
                    vals = plsc.load_gather(ch, [rows, jbase + d])
                    tr[d, pl.ds(g8 * SUB, SUB)] = vals

        def fire_store(g, tr, semt):
            h = hh * h_half + g // runs
            bcol = b0 + (g % runs) * CHUNK
            pltpu.async_copy(tr, out_hbm.at[h, :, pl.ds(bcol, CHUNK)], semt)

        def wait_store(tr, semt):
            pltpu.make_async_copy(tr, out_hbm.at[0, :, pl.ds(0, CHUNK)],
                                  semt).wait()

        compute_sidx(0, sb0)
        fire_gather(sb0, ch0, semg0)

        def body(gg, carry):
            g0 = 2 * gg

            compute_sidx(g0 + 1, sb1)
            fire_gather(sb1, ch1, semg1)
            wait_gather(ch0, semg0)

            @pl.when(gg > 0)
            def _():
                wait_store(tr0, semt0)

            extract(g0, ch0, tr0)
            fire_store(g0, tr0, semt0)

            @pl.when(gg < n_chunks // 2 - 1)
            def _():
                compute_sidx(g0 + 2, sb0)
                fire_gather(sb0, ch0, semg0)

            wait_gather(ch1, semg1)

            @pl.when(gg > 0)
            def _():
                wait_store(tr1, semt1)

            extract(g0 + 1, ch1, tr1)
            fire_store(g0 + 1, tr1, semt1)
            return carry

        lax.fori_loop(0, n_chunks // 2, body, 0)
        wait_store(tr0, semt0)
        wait_store(tr1, semt1)

    return lookup


def kernel(inputs, table):
    ids = inputs
    if ids.ndim > 2:
        ids = jnp.squeeze(ids, axis=-1)
    batch, hist = ids.shape
    vocab, dim = table.shape
    h_half = hist // 2
    n_bw = batch // B_PER_W
    ids_w = (jnp.transpose(ids)
             .reshape(2, h_half, n_bw, B_PER_W)
             .transpose(0, 2, 1, 3)
             .reshape(batch * hist))
    table_sr = table.reshape(vocab // 4, dim * 4)
    out_t = _make_lookup(vocab, batch, hist, dim)(table_sr, ids_w)
    return jnp.transpose(out_t, (2, 0, 1))


# speedup vs baseline: 1.0300x; 1.0300x over previous
import functools

import jax
import jax.numpy as jnp
from jax import lax
from jax.experimental import pallas as pl
from jax.experimental.pallas import tpu as pltpu
from jax.experimental.pallas import tpu_sc as plsc

LANES = 128
CHUNK = 256
B_PER_W = 1024
SUB = 16


@functools.lru_cache(maxsize=None)
def _make_lookup(vocab, batch, hist, dim):
    info = plsc.get_sparse_core_info()
    nc, ns = info.num_cores, info.num_subcores
    h_half = hist // 2
    n_bw = batch // B_PER_W
    runs = B_PER_W // CHUNK
    n_chunks = h_half * runs
    per_w = h_half * B_PER_W
    sr = dim * 4
    streams = CHUNK // LANES

    mesh = plsc.VectorSubcoreMesh(core_axis_name="c", subcore_axis_name="s")

    @functools.partial(
        pl.kernel,
        mesh=mesh,
        out_type=jax.ShapeDtypeStruct((hist, dim, batch), jnp.float32),
        scratch_types=[
            pltpu.VMEM((per_w,), jnp.int32),
            pltpu.VMEM((streams, LANES), jnp.int32),
            pltpu.VMEM((streams, LANES), jnp.int32),
            pltpu.VMEM((CHUNK, sr), jnp.float32),
            pltpu.VMEM((CHUNK, sr), jnp.float32),
            pltpu.VMEM((dim, CHUNK), jnp.float32),
            pltpu.VMEM((dim, CHUNK), jnp.float32),
            pltpu.SemaphoreType.DMA,
            pltpu.SemaphoreType.DMA,
            pltpu.SemaphoreType.DMA,
            pltpu.SemaphoreType.DMA,
        ],
        compiler_params=pltpu.CompilerParams(use_tc_tiling_on_sc=True,
                                             needs_layout_passes=False),
    )
    def lookup(table_hbm, ids_hbm, out_hbm, idx_v, sb0, sb1, ch0, ch1,
               tr0, tr1, semg0, semg1, semt0, semt1):
        wid = lax.axis_index("s") * nc + lax.axis_index("c")
        hh = wid // n_bw
        bb = wid % n_bw
        b0 = bb * B_PER_W

        pltpu.sync_copy(ids_hbm.at[pl.ds(wid * per_w, per_w)], idx_v)

        def compute_sidx(g, sb):
            p0 = g * CHUNK
            for j in range(streams):
                for g8 in range(LANES // SUB):
                    v = idx_v[pl.ds(p0 + j * LANES + g8 * SUB, SUB)]
                    sb[j, pl.ds(g8 * SUB, SUB)] = v >> 2

        def fire_gather(sb, ch, semg):
            for j in range(streams):
                pltpu.async_copy(table_hbm.at[sb.at[j]],
                                 ch.at[pl.ds(j * LANES, LANES)], semg)

        def wait_gather(ch, semg):
            pltpu.make_async_copy(table_hbm.at[pl.ds(0, CHUNK)], ch,
                                  semg).wait()

        def extract(g, ch, tr):
            p0 = g * CHUNK
            ivec = lax.iota(jnp.int32, SUB)

            @plsc.parallel_loop(0, CHUNK // SUB, unroll=4)
            def _(g8):
                v = idx_v[pl.ds(p0 + g8 * SUB, SUB)]
                jbase = (v & 3) * dim
                rows = ivec + g8 * SUB
                for d in range(dim):
                    vals = plsc.load_gather(ch, [rows, jbase + d])
                    tr[d, pl.ds(g8 * SUB, SUB)] = vals

        def fire_store(g, tr, semt):
            h = hh * h_half + g // runs
            bcol = b0 + (g % runs) * CHUNK
            pltpu.async_copy(tr, out_hbm.at[h, :, pl.ds(bcol, CHUNK)], semt)

        def wait_store(tr, semt):
            pltpu.make_async_copy(tr, out_hbm.at[0, :, pl.ds(0, CHUNK)],
                                  semt).wait()

        compute_sidx(0, sb0)
        fire_gather(sb0, ch0, semg0)

        def body(gg, carry):
            g0 = 2 * gg

            compute_sidx(g0 + 1, sb1)
            fire_gather(sb1, ch1, semg1)
            wait_gather(ch0, semg0)

            @pl.when(gg > 0)
            def _():
                wait_store(tr0, semt0)

            extract(g0, ch0, tr0)
            fire_store(g0, tr0, semt0)

            @pl.when(gg < n_chunks // 2 - 1)
            def _():
                compute_sidx(g0 + 2, sb0)
                fire_gather(sb0, ch0, semg0)

            wait_gather(ch1, semg1)

            @pl.when(gg > 0)
            def _():
                wait_store(tr1, semt1)

            extract(g0 + 1, ch1, tr1)
            fire_store(g0 + 1, tr1, semt1)
            return carry

        lax.fori_loop(0, n_chunks // 2, body, 0)
        wait_store(tr0, semt0)
        wait_store(tr1, semt1)

    return lookup


def kernel(inputs, table):
    ids = inputs
    if ids.ndim > 2:
        ids = jnp.squeeze(ids, axis=-1)
    batch, hist = ids.shape
    vocab, dim = table.shape
    h_half = hist // 2
    n_bw = batch // B_PER_W
    ids_w = (jnp.transpose(ids)
             .reshape(2, h_half, n_bw, B_PER_W)
             .transpose(0, 2, 1, 3)
             .reshape(batch * hist))
    table_sr = table.reshape(vocab // 4, dim * 4)
    out_t = _make_lookup(vocab, batch, hist, dim)(table_sr, ids_w)
    return jnp.transpose(out_t, (2, 0, 1))
